# baseline (speedup 1.0000x reference)
import functools

import jax
import jax.numpy as jnp
from jax import lax
from jax.experimental import pallas as pl
from jax.experimental.pallas import tpu as pltpu
from jax.experimental.pallas import tpu_sc as plsc

_V = 1000
_NTOT = 1024 * 50
_NC = 2
_NS = 16
_PER_C = _NTOT // _NC
_PER_W = _PER_C // _NS
_CH = 16
_NCH = _PER_W // _CH
_LAG = 16
_ROWS_PER_S = 62


def _sc_gather(table, idx_flat):
  mesh = plsc.VectorSubcoreMesh(core_axis_name="c", subcore_axis_name="s")

  @functools.partial(
      pl.kernel,
      mesh=mesh,
      out_type=[
          jax.ShapeDtypeStruct((_PER_C, _V), jnp.float32),
          jax.ShapeDtypeStruct((_PER_C, _V), jnp.float32),
      ],
      compiler_params=pltpu.CompilerParams(use_tc_tiling_on_sc=False),
      scratch_types=[
          pltpu.VMEM_SHARED((_V, _V), jnp.float32),
          pltpu.VMEM((_PER_W,), jnp.int32),
          pltpu.SemaphoreType.DMA,
      ],
  )
  def k(table_hbm, idx_hbm, out0_hbm, out1_hbm, table_sp, idx_v, sem):
    sid = lax.axis_index("s")
    cid = lax.axis_index("c")

    r0 = sid * _ROWS_PER_S
    pltpu.sync_copy(table_hbm.at[pl.ds(r0, _ROWS_PER_S)],
                    table_sp.at[pl.ds(r0, _ROWS_PER_S)])

    @pl.when(sid == 0)
    def _tail():
      pltpu.sync_copy(table_hbm.at[pl.ds(_ROWS_PER_S * 16, 8)],
                      table_sp.at[pl.ds(_ROWS_PER_S * 16, 8)])

    pltpu.sync_copy(idx_hbm.at[pl.ds(cid * _PER_C + sid * _PER_W, _PER_W)],
                    idx_v)
    plsc.subcore_barrier()

    def run(out_hbm):
      base = sid * _PER_W

      def issue_chunk(c):
        chunk = idx_v[pl.ds(c * _CH, _CH)]
        for l in range(_CH):
          row = chunk[l]
          pltpu.async_copy(table_sp.at[row], out_hbm.at[base + c * _CH + l],
                           sem)

      def wait_one():
        pltpu.make_async_copy(table_sp.at[0], out_hbm.at[0], sem).wait()

      def prime(c, carry):
        issue_chunk(c)
        return carry

      lax.fori_loop(0, _LAG, prime, 0)

      def body(c, carry):
        for _ in range(_CH):
          wait_one()
        issue_chunk(c)
        return carry

      lax.fori_loop(_LAG, _NCH, body, 0)

      def drain(i, carry):
        wait_one()
        return carry

      lax.fori_loop(0, _LAG * _CH, drain, 0)

    @pl.when(cid == 0)
    def _c0():
      run(out0_hbm)

    @pl.when(cid == 1)
    def _c1():
      run(out1_hbm)

  return k(table, idx_flat)


def kernel(x, prob):
  idx = x.reshape(-1).astype(jnp.int32)
  out0, out1 = _sc_gather(prob, idx)
  out = jnp.concatenate([out0, out1], axis=0)
  return out.reshape(x.shape[0], x.shape[1], _V)


# speedup vs baseline: 1.3143x; 1.3143x over previous
import functools

import jax
import jax.numpy as jnp
from jax import lax
from jax.experimental import pallas as pl
from jax.experimental.pallas import tpu as pltpu
from jax.experimental.pallas import tpu_sc as plsc

_V = 1000
_NTOT = 1024 * 50
_NW = 32
_PER_W = _NTOT // _NW
_C = 32
_NCHUNK = _PER_W // _C
_ROWS_PER_S = 62


def _sc_gather(table, idx_flat):
  mesh = plsc.VectorSubcoreMesh(core_axis_name="c", subcore_axis_name="s")

  @functools.partial(
      pl.kernel,
      mesh=mesh,
      out_type=jax.ShapeDtypeStruct((_NTOT, _V), jnp.float32),
      compiler_params=pltpu.CompilerParams(use_tc_tiling_on_sc=False),
      scratch_types=[
          pltpu.VMEM_SHARED((_V, _V), jnp.float32),
          pltpu.VMEM((_PER_W,), jnp.int32),
          pltpu.VMEM((_C, _V), jnp.float32),
          pltpu.VMEM((_C, _V), jnp.float32),
          pltpu.SemaphoreType.DMA,
          pltpu.SemaphoreType.DMA,
          pltpu.SemaphoreType.DMA,
          pltpu.SemaphoreType.DMA,
      ],
  )
  def k(table_hbm, idx_hbm, out_hbm, table_sp, idx_v, rows0, rows1,
        gsem0, gsem1, ssem0, ssem1):
    sid = lax.axis_index("s")
    wid = sid * 2 + lax.axis_index("c")
    base = wid * _PER_W

    r0 = sid * _ROWS_PER_S
    pltpu.sync_copy(table_hbm.at[pl.ds(r0, _ROWS_PER_S)],
                    table_sp.at[pl.ds(r0, _ROWS_PER_S)])

    @pl.when(sid == 0)
    def _tail_rows():
      pltpu.sync_copy(table_hbm.at[pl.ds(_ROWS_PER_S * 16, 8)],
                      table_sp.at[pl.ds(_ROWS_PER_S * 16, 8)])

    pltpu.sync_copy(idx_hbm.at[pl.ds(base, _PER_W)], idx_v)
    plsc.subcore_barrier()

    def start_gather(g, buf, sem):
      pltpu.async_copy(table_sp.at[idx_v.at[pl.ds(g * _C, _C)]], buf, sem)

    def wait_gather(buf, sem):
      pltpu.make_async_copy(table_sp.at[idx_v.at[pl.ds(0, _C)]], buf,
                            sem).wait()

    def start_scatter(g, buf, sem):
      pltpu.async_copy(buf, out_hbm.at[pl.ds(base + g * _C, _C)], sem)

    def wait_scatter(buf, sem):
      pltpu.make_async_copy(buf, out_hbm.at[pl.ds(base, _C)], sem).wait()

    start_gather(0, rows0, gsem0)
    start_gather(1, rows1, gsem1)

    def body(p, carry):
      g = 2 * p
      wait_gather(rows0, gsem0)
      start_scatter(g, rows0, ssem0)
      wait_gather(rows1, gsem1)
      start_scatter(g + 1, rows1, ssem1)
      wait_scatter(rows0, ssem0)
      start_gather(g + 2, rows0, gsem0)
      wait_scatter(rows1, ssem1)
      start_gather(g + 3, rows1, gsem1)
      return carry

    lax.fori_loop(0, _NCHUNK // 2 - 1, body, 0)

    wait_gather(rows0, gsem0)
    start_scatter(_NCHUNK - 2, rows0, ssem0)
    wait_gather(rows1, gsem1)
    start_scatter(_NCHUNK - 1, rows1, ssem1)
    wait_scatter(rows0, ssem0)
    wait_scatter(rows1, ssem1)

  return k(table, idx_flat)


def kernel(x, prob):
  idx = x.reshape(-1).astype(jnp.int32)
  out = _sc_gather(prob, idx)
  return out.reshape(x.shape[0], x.shape[1], _V)


# speedup vs baseline: 1.3682x; 1.0409x over previous
import functools

import jax
import jax.numpy as jnp
from jax import lax
from jax.experimental import pallas as pl
from jax.experimental.pallas import tpu as pltpu
from jax.experimental.pallas import tpu_sc as plsc

_V = 1000
_NTOT = 1024 * 50
_NW = 32
_PER_W = _NTOT // _NW
_C = 32
_NSCH = 24
_DBASE = _NSCH * _C
_DCH = 16
_NDCH = (_PER_W - _DBASE) // _DCH
_DLAG = 16
_DPER = 3
_ROWS_PER_S = 62


def _sc_gather(table, idx_flat):
  mesh = plsc.VectorSubcoreMesh(core_axis_name="c", subcore_axis_name="s")

  @functools.partial(
      pl.kernel,
      mesh=mesh,
      out_type=jax.ShapeDtypeStruct((_NTOT, _V), jnp.float32),
      compiler_params=pltpu.CompilerParams(use_tc_tiling_on_sc=False),
      scratch_types=[
          pltpu.VMEM_SHARED((_V, _V), jnp.float32),
          pltpu.VMEM((_PER_W,), jnp.int32),
          pltpu.VMEM((_C, _V), jnp.float32),
          pltpu.VMEM((_C, _V), jnp.float32),
          pltpu.SemaphoreType.DMA,
          pltpu.SemaphoreType.DMA,
          pltpu.SemaphoreType.DMA,
          pltpu.SemaphoreType.DMA,
          pltpu.SemaphoreType.DMA,
      ],
  )
  def k(table_hbm, idx_hbm, out_hbm, table_sp, idx_v, rows0, rows1,
        gsem0, gsem1, ssem0, ssem1, dsem):
    sid = lax.axis_index("s")
    wid = sid * 2 + lax.axis_index("c")
    base = wid * _PER_W

    r0 = sid * _ROWS_PER_S
    pltpu.sync_copy(table_hbm.at[pl.ds(r0, _ROWS_PER_S)],
                    table_sp.at[pl.ds(r0, _ROWS_PER_S)])

    @pl.when(sid == 0)
    def _tail_rows():
      pltpu.sync_copy(table_hbm.at[pl.ds(_ROWS_PER_S * 16, 8)],
                      table_sp.at[pl.ds(_ROWS_PER_S * 16, 8)])

    pltpu.sync_copy(idx_hbm.at[pl.ds(base, _PER_W)], idx_v)
    plsc.subcore_barrier()

    def start_gather(g, buf, sem):
      pltpu.async_copy(table_sp.at[idx_v.at[pl.ds(g * _C, _C)]], buf, sem)

    def wait_gather(buf, sem):
      pltpu.make_async_copy(table_sp.at[idx_v.at[pl.ds(0, _C)]], buf,
                            sem).wait()

    def start_scatter(g, buf, sem):
      pltpu.async_copy(buf, out_hbm.at[pl.ds(base + g * _C, _C)], sem)

    def wait_scatter(buf, sem):
      pltpu.make_async_copy(buf, out_hbm.at[pl.ds(base, _C)], sem).wait()

    def issue_dchunk(d):
      chunk = idx_v[pl.ds(_DBASE + d * _DCH, _DCH)]
      for l in range(_DCH):
        row = chunk[l]
        pltpu.async_copy(table_sp.at[row],
                         out_hbm.at[base + _DBASE + d * _DCH + l], dsem)

    def wait_d():
      pltpu.make_async_copy(table_sp.at[0], out_hbm.at[0], dsem).wait()

    def dprime(d, carry):
      issue_dchunk(d)
      return carry

    lax.fori_loop(0, _DLAG, dprime, 0)

    start_gather(0, rows0, gsem0)
    start_gather(1, rows1, gsem1)

    def body(p, carry):
      g = 2 * p
      wait_gather(rows0, gsem0)
      start_scatter(g, rows0, ssem0)
      wait_gather(rows1, gsem1)
      start_scatter(g + 1, rows1, ssem1)
      wait_scatter(rows0, ssem0)
      start_gather(g + 2, rows0, gsem0)
      wait_scatter(rows1, ssem1)
      start_gather(g + 3, rows1, gsem1)
      for q in range(_DPER):
        for _ in range(_DCH):
          wait_d()
        issue_dchunk(_DLAG + _DPER * p + q)
      return carry

    lax.fori_loop(0, _NSCH // 2 - 1, body, 0)

    wait_gather(rows0, gsem0)
    start_scatter(_NSCH - 2, rows0, ssem0)
    wait_gather(rows1, gsem1)
    start_scatter(_NSCH - 1, rows1, ssem1)
    wait_scatter(rows0, ssem0)
    wait_scatter(rows1, ssem1)

    def dfeed(d, carry):
      for _ in range(_DCH):
        wait_d()
      issue_dchunk(d)
      return carry

    lax.fori_loop(_DLAG + _DPER * (_NSCH // 2 - 1), _NDCH, dfeed, 0)

    def ddrain(i, carry):
      wait_d()
      return carry

    lax.fori_loop(0, _DLAG * _DCH, ddrain, 0)

  return k(table, idx_flat)


def kernel(x, prob):
  idx = x.reshape(-1).astype(jnp.int32)
  out = _sc_gather(prob, idx)
  return out.reshape(x.shape[0], x.shape[1], _V)
